# Initial kernel scaffold; baseline (speedup 1.0000x reference)
import jax
import jax.numpy as jnp
from jax.experimental import pallas as pl


def kernel(x, Wq, Wk, Wv, Wo, Wknn_key, Wknn_proj, b_knn_proj, Wgate, b_gate):
    raise NotImplementedError("write your pallas kernel here")


# speedup vs baseline: 1.4912x; 1.4912x over previous
import jax
import jax.numpy as jnp
import numpy as np
from jax.experimental import pallas as pl
from jax.experimental.pallas import tpu as pltpu

D = 768
H = 12
HD = D // H


def _dot(a, b, dims):
    return jax.lax.dot_general(a, b, (dims, ((), ())),
                               preferred_element_type=jnp.float32)


def _mha_gate_kernel(x_ref, wq_ref, wk_ref, wv_ref, wo_ref, wg_ref,
                     bknn_ref, bg_ref, out_ref):
    h = pl.program_id(0)
    x = x_ref[...]
    q = _dot(x, wq_ref[...], ((1,), (1,)))
    k = _dot(x, wk_ref[...], ((1,), (1,)))
    v = _dot(x, wv_ref[...], ((1,), (1,)))

    s = _dot(q, k, ((1,), (1,))) * (1.0 / np.sqrt(HD))
    n = s.shape[0]
    row = jax.lax.broadcasted_iota(jnp.int32, (n, n), 0)
    col = jax.lax.broadcasted_iota(jnp.int32, (n, n), 1)
    s = jnp.where(col <= row, s, -jnp.inf)
    m = jnp.max(s, axis=1, keepdims=True)
    p = jnp.exp(s - m)
    l = jnp.sum(p, axis=1, keepdims=True)
    attn = _dot(p / l, v, ((1,), (0,)))
    o = _dot(attn, wo_ref[...], ((1,), (0,)))

    @pl.when(h == 0)
    def _():
        out_ref[...] = o

    @pl.when(h > 0)
    def _():
        out_ref[...] += o

    @pl.when(h == H - 1)
    def _():
        a = out_ref[...]
        wg = wg_ref[...]
        wg1 = wg[:, :D]
        wg2 = wg[:, D:]
        bknn = bknn_ref[...]
        c = jnp.sum(bknn * wg2) + bg_ref[0, 0]
        g_logit = jnp.sum(a * wg1, axis=1, keepdims=True) + c
        g = jax.nn.sigmoid(g_logit)
        out_ref[...] = g * a + (1.0 - g) * bknn


def kernel(x, Wq, Wk, Wv, Wo, Wknn_key, Wknn_proj, b_knn_proj, Wgate, b_gate):
    b, s, d = x.shape
    x2 = x.reshape(s, d)
    bknn = b_knn_proj.reshape(1, d)
    bg = b_gate.reshape(1, 1)

    out = pl.pallas_call(
        _mha_gate_kernel,
        grid=(H,),
        in_specs=[
            pl.BlockSpec((s, d), lambda h: (0, 0)),
            pl.BlockSpec((HD, d), lambda h: (h, 0)),
            pl.BlockSpec((HD, d), lambda h: (h, 0)),
            pl.BlockSpec((HD, d), lambda h: (h, 0)),
            pl.BlockSpec((HD, d), lambda h: (h, 0)),
            pl.BlockSpec((1, 2 * d), lambda h: (0, 0)),
            pl.BlockSpec((1, d), lambda h: (0, 0)),
            pl.BlockSpec((1, 1), lambda h: (0, 0)),
        ],
        out_specs=pl.BlockSpec((s, d), lambda h: (0, 0)),
        out_shape=jax.ShapeDtypeStruct((s, d), jnp.float32),
        compiler_params=pltpu.CompilerParams(
            dimension_semantics=("arbitrary",),
        ),
    )(x2, Wq, Wk, Wv, Wo.T, Wgate, bknn, bg)
    return out.reshape(b, s, d)


# speedup vs baseline: 2.0099x; 1.3478x over previous
import jax
import jax.numpy as jnp
import numpy as np
from jax.experimental import pallas as pl
from jax.experimental.pallas import tpu as pltpu

D = 768
H = 12
HD = D // H


def _dot(a, b, dims):
    return jax.lax.dot_general(a, b, (dims, ((), ())),
                               preferred_element_type=jnp.float32)


QBLK = 256


def _mha_gate_kernel(x_ref, wq_ref, wk_ref, wv_ref, wo_ref, wg_ref,
                     bknn_ref, bg_ref, out_ref):
    h = pl.program_id(0)
    x = x_ref[...]
    scale = 1.0 / np.sqrt(HD)
    q = _dot(x, wq_ref[...], ((1,), (1,))) * scale
    k = _dot(x, wk_ref[...], ((1,), (1,)))
    v = _dot(x, wv_ref[...], ((1,), (1,)))

    n = x.shape[0]
    nblk = n // QBLK
    row = jax.lax.broadcasted_iota(jnp.int32, (QBLK, QBLK), 0)
    col = jax.lax.broadcasted_iota(jnp.int32, (QBLK, QBLK), 1)
    diag_mask = col <= row

    attn_blocks = []
    for i in range(nblk):
        qi = q[i * QBLK:(i + 1) * QBLK, :]
        kd = k[i * QBLK:(i + 1) * QBLK, :]
        vd = v[i * QBLK:(i + 1) * QBLK, :]
        s_diag = _dot(qi, kd, ((1,), (1,)))
        s_diag = jnp.where(diag_mask, s_diag, -jnp.inf)
        if i == 0:
            m = jnp.max(s_diag, axis=1, keepdims=True)
            p_diag = jnp.exp(s_diag - m)
            l = jnp.sum(p_diag, axis=1, keepdims=True)
            acc = _dot(p_diag, vd, ((1,), (0,)))
        else:
            kp = k[:i * QBLK, :]
            vp = v[:i * QBLK, :]
            s_prev = _dot(qi, kp, ((1,), (1,)))
            m = jnp.maximum(jnp.max(s_prev, axis=1, keepdims=True),
                            jnp.max(s_diag, axis=1, keepdims=True))
            p_prev = jnp.exp(s_prev - m)
            p_diag = jnp.exp(s_diag - m)
            l = (jnp.sum(p_prev, axis=1, keepdims=True) +
                 jnp.sum(p_diag, axis=1, keepdims=True))
            acc = (_dot(p_prev, vp, ((1,), (0,))) +
                   _dot(p_diag, vd, ((1,), (0,))))
        attn_blocks.append(acc / l)

    attn = jnp.concatenate(attn_blocks, axis=0)
    o = _dot(attn, wo_ref[...], ((1,), (0,)))

    @pl.when(h == 0)
    def _():
        out_ref[...] = o

    @pl.when(h > 0)
    def _():
        out_ref[...] += o

    @pl.when(h == H - 1)
    def _():
        a = out_ref[...]
        wg = wg_ref[...]
        wg1 = wg[:, :D]
        wg2 = wg[:, D:]
        bknn = bknn_ref[...]
        c = jnp.sum(bknn * wg2) + bg_ref[0, 0]
        g_logit = jnp.sum(a * wg1, axis=1, keepdims=True) + c
        g = jax.nn.sigmoid(g_logit)
        out_ref[...] = g * a + (1.0 - g) * bknn


def kernel(x, Wq, Wk, Wv, Wo, Wknn_key, Wknn_proj, b_knn_proj, Wgate, b_gate):
    b, s, d = x.shape
    x2 = x.reshape(s, d)
    bknn = b_knn_proj.reshape(1, d)
    bg = b_gate.reshape(1, 1)

    out = pl.pallas_call(
        _mha_gate_kernel,
        grid=(H,),
        in_specs=[
            pl.BlockSpec((s, d), lambda h: (0, 0)),
            pl.BlockSpec((HD, d), lambda h: (h, 0)),
            pl.BlockSpec((HD, d), lambda h: (h, 0)),
            pl.BlockSpec((HD, d), lambda h: (h, 0)),
            pl.BlockSpec((HD, d), lambda h: (h, 0)),
            pl.BlockSpec((1, 2 * d), lambda h: (0, 0)),
            pl.BlockSpec((1, d), lambda h: (0, 0)),
            pl.BlockSpec((1, 1), lambda h: (0, 0)),
        ],
        out_specs=pl.BlockSpec((s, d), lambda h: (0, 0)),
        out_shape=jax.ShapeDtypeStruct((s, d), jnp.float32),
        compiler_params=pltpu.CompilerParams(
            dimension_semantics=("arbitrary",),
        ),
    )(x2, Wq, Wk, Wv, Wo.T, Wgate, bknn, bg)
    return out.reshape(b, s, d)


# speedup vs baseline: 2.7836x; 1.3850x over previous
import jax
import jax.numpy as jnp
import numpy as np
from jax.experimental import pallas as pl
from jax.experimental.pallas import tpu as pltpu

D = 768
H = 12
HD = D // H
NPAIR = H // 2
QBLK = 256


def _dot(a, b, dims):
    return jax.lax.dot_general(a, b, (dims, ((), ())),
                               preferred_element_type=jnp.float32)


def _causal_head_attn(q, k, v):
    n = q.shape[0]
    nblk = n // QBLK
    row = jax.lax.broadcasted_iota(jnp.int32, (QBLK, QBLK), 0)
    col = jax.lax.broadcasted_iota(jnp.int32, (QBLK, QBLK), 1)
    diag_mask = col <= row

    blocks = []
    for i in range(nblk):
        qi = q[i * QBLK:(i + 1) * QBLK, :]
        kd = k[i * QBLK:(i + 1) * QBLK, :]
        vd = v[i * QBLK:(i + 1) * QBLK, :]
        s_diag = _dot(qi, kd, ((1,), (1,)))
        s_diag = jnp.where(diag_mask, s_diag, -jnp.inf)
        if i == 0:
            m = jnp.max(s_diag, axis=1, keepdims=True)
            p_diag = jnp.exp(s_diag - m)
            l = jnp.sum(p_diag, axis=1, keepdims=True)
            acc = _dot(p_diag, vd, ((1,), (0,)))
        else:
            kp = k[:i * QBLK, :]
            vp = v[:i * QBLK, :]
            s_prev = _dot(qi, kp, ((1,), (1,)))
            m = jnp.maximum(jnp.max(s_prev, axis=1, keepdims=True),
                            jnp.max(s_diag, axis=1, keepdims=True))
            p_prev = jnp.exp(s_prev - m)
            p_diag = jnp.exp(s_diag - m)
            l = (jnp.sum(p_prev, axis=1, keepdims=True) +
                 jnp.sum(p_diag, axis=1, keepdims=True))
            acc = (_dot(p_prev, vp, ((1,), (0,))) +
                   _dot(p_diag, vd, ((1,), (0,))))
        blocks.append(acc / l)
    return jnp.concatenate(blocks, axis=0)


def _mha_gate_kernel(x_ref, wall_ref, wot_ref, wg_ref, bknn_ref, bg_ref,
                     out_ref, qkv_ref):
    t = pl.program_id(0)

    @pl.when(t == 0)
    def _():
        qkv_ref[...] = _dot(x_ref[...], wall_ref[...], ((1,), (1,)))

    @pl.when(t > 0)
    def _():
        pair = t - 1
        base = pair * (2 * HD)
        scale = 1.0 / np.sqrt(HD)
        q2 = qkv_ref[:, pl.ds(base, 2 * HD)] * scale
        k2 = qkv_ref[:, pl.ds(D + base, 2 * HD)]
        v2 = qkv_ref[:, pl.ds(2 * D + base, 2 * HD)]

        a0 = _causal_head_attn(q2[:, :HD], k2[:, :HD], v2[:, :HD])
        a1 = _causal_head_attn(q2[:, HD:], k2[:, HD:], v2[:, HD:])
        attn_pair = jnp.concatenate([a0, a1], axis=1)
        wot_pair = wot_ref[pl.ds(base, 2 * HD), :]
        o = _dot(attn_pair, wot_pair, ((1,), (0,)))

        @pl.when(t == 1)
        def _():
            out_ref[...] = o

        @pl.when(t > 1)
        def _():
            out_ref[...] += o

        @pl.when(t == NPAIR)
        def _():
            a = out_ref[...]
            wg = wg_ref[...]
            wg1 = wg[:, :D]
            wg2 = wg[:, D:]
            bknn = bknn_ref[...]
            c = jnp.sum(bknn * wg2) + bg_ref[0, 0]
            g_logit = jnp.sum(a * wg1, axis=1, keepdims=True) + c
            g = jax.nn.sigmoid(g_logit)
            out_ref[...] = g * a + (1.0 - g) * bknn


def kernel(x, Wq, Wk, Wv, Wo, Wknn_key, Wknn_proj, b_knn_proj, Wgate, b_gate):
    b, s, d = x.shape
    x2 = x.reshape(s, d)
    w_all = jnp.concatenate([Wq, Wk, Wv], axis=0)
    bknn = b_knn_proj.reshape(1, d)
    bg = b_gate.reshape(1, 1)

    out = pl.pallas_call(
        _mha_gate_kernel,
        grid=(1 + NPAIR,),
        in_specs=[
            pl.BlockSpec((s, d), lambda t: (0, 0)),
            pl.BlockSpec((3 * d, d), lambda t: (0, 0)),
            pl.BlockSpec((d, d), lambda t: (0, 0)),
            pl.BlockSpec((1, 2 * d), lambda t: (0, 0)),
            pl.BlockSpec((1, d), lambda t: (0, 0)),
            pl.BlockSpec((1, 1), lambda t: (0, 0)),
        ],
        out_specs=pl.BlockSpec((s, d), lambda t: (0, 0)),
        out_shape=jax.ShapeDtypeStruct((s, d), jnp.float32),
        scratch_shapes=[pltpu.VMEM((s, 3 * d), jnp.float32)],
        compiler_params=pltpu.CompilerParams(
            dimension_semantics=("arbitrary",),
        ),
    )(x2, w_all, Wo.T, Wgate, bknn, bg)
    return out.reshape(b, s, d)
